# baseline (speedup 1.0000x reference)
import functools

import jax
import jax.numpy as jnp
from jax import lax
from jax.experimental import pallas as pl
from jax.experimental.pallas import tpu as pltpu
from jax.experimental.pallas import tpu_sc as plsc

ACTION_DIM = 4096
TARGET_ID = 123
TARGET_VAL = 100.0

_INFO = plsc.get_sparse_core_info()
_NC = 1
_NS = 1
_L = _INFO.num_lanes
_NW = _NC * _NS
_CHUNK = ACTION_DIM // _NW

_MESH = plsc.VectorSubcoreMesh(
    core_axis_name="c", subcore_axis_name="s",
    num_cores=_NC, num_subcores=_NS,
)


@functools.partial(
    pl.kernel,
    mesh=_MESH,
    out_type=jax.ShapeDtypeStruct((ACTION_DIM,), jnp.float32),
    scratch_types=[pltpu.VMEM((_CHUNK,), jnp.float32)],
)
def _mock_q(out_hbm, buf):
    wid = lax.axis_index("s") * _NC + lax.axis_index("c")
    base = wid * _CHUNK
    lanes = lax.iota(jnp.int32, _L)
    for i in range(_CHUNK // _L):
        g = base + i * _L + lanes
        buf[pl.ds(i * _L, _L)] = jnp.where(
            g == TARGET_ID, jnp.float32(TARGET_VAL), jnp.float32(0.0)
        )
    pltpu.sync_copy(buf, out_hbm.at[pl.ds(base, _CHUNK)])


def kernel(x, player_side=1):
    del x, player_side
    return _mock_q().reshape(1, ACTION_DIM)


# speedup vs baseline: 30.9559x; 30.9559x over previous
import jax
import jax.numpy as jnp
from jax import lax
from jax.experimental import pallas as pl

ACTION_DIM = 4096
TARGET_ID = 123
TARGET_VAL = 100.0


def _mock_q_body(o_ref):
    col = lax.broadcasted_iota(jnp.int32, (1, ACTION_DIM), dimension=1)
    o_ref[...] = jnp.where(
        col == TARGET_ID, jnp.float32(TARGET_VAL), jnp.float32(0.0)
    )


def kernel(x, player_side=1):
    del x, player_side
    return pl.pallas_call(
        _mock_q_body,
        out_shape=jax.ShapeDtypeStruct((1, ACTION_DIM), jnp.float32),
    )()
